# baseline (speedup 1.0000x reference)
import jax
import jax.numpy as jnp
from jax.experimental import pallas as pl
from jax.experimental.pallas import tpu as pltpu


def _pos_body(col_ref, row_ref, out_ref):
    nb, yb, w = out_ref.shape[:3]
    d = col_ref.shape[1]
    col = col_ref[:w, :]
    row = row_ref[:yb, :]
    colb = jnp.broadcast_to(col[None, None, :, :], (nb, yb, w, d))
    rowb = jnp.broadcast_to(row[None, :, None, :], (nb, yb, w, d))
    out_ref[...] = jnp.concatenate([colb, rowb], axis=-1)


@jax.jit
def kernel(pixel_values, row_embeddings, column_embeddings):
    b = pixel_values.shape[0]
    h, w = pixel_values.shape[-2], pixel_values.shape[-1]
    d = column_embeddings.shape[-1]
    nb = 4

    out = pl.pallas_call(
        _pos_body,
        grid=(b // nb,),
        in_specs=[
            pl.BlockSpec((w, d), lambda i: (0, 0)),
            pl.BlockSpec((h, d), lambda i: (0, 0)),
        ],
        out_specs=pl.BlockSpec((nb, h, w, 2 * d), lambda i: (i, 0, 0, 0)),
        out_shape=jax.ShapeDtypeStruct((b, h, w, 2 * d), jnp.float32),
        compiler_params=pltpu.CompilerParams(
            dimension_semantics=("parallel",),
        ),
    )(column_embeddings, row_embeddings)
    return out.transpose(0, 3, 1, 2)


# speedup vs baseline: 1.1499x; 1.1499x over previous
import jax
import jax.numpy as jnp
from jax.experimental import pallas as pl
from jax.experimental.pallas import tpu as pltpu


def _pos_body(col_ref, row_ref, out_ref):
    nb, yb, w = out_ref.shape[:3]
    d = col_ref.shape[1]
    col = col_ref[:w, :]
    row = row_ref[:yb, :]
    colb = jnp.broadcast_to(col[None, None, :, :], (nb, yb, w, d))
    rowb = jnp.broadcast_to(row[None, :, None, :], (nb, yb, w, d))
    out_ref[...] = jnp.concatenate([colb, rowb], axis=-1)


@jax.jit
def kernel(pixel_values, row_embeddings, column_embeddings):
    b = pixel_values.shape[0]
    h, w = pixel_values.shape[-2], pixel_values.shape[-1]
    d = column_embeddings.shape[-1]
    nb = 2

    out = pl.pallas_call(
        _pos_body,
        grid=(b // nb,),
        in_specs=[
            pl.BlockSpec((w, d), lambda i: (0, 0)),
            pl.BlockSpec((h, d), lambda i: (0, 0)),
        ],
        out_specs=pl.BlockSpec((nb, h, w, 2 * d), lambda i: (i, 0, 0, 0)),
        out_shape=jax.ShapeDtypeStruct((b, h, w, 2 * d), jnp.float32),
        compiler_params=pltpu.CompilerParams(
            dimension_semantics=("parallel",),
        ),
    )(column_embeddings, row_embeddings)
    return out.transpose(0, 3, 1, 2)
